# baseline (speedup 1.0000x reference)
import jax
import jax.numpy as jnp
from jax.experimental import pallas as pl

R, C = 128, 32768
BR = 32


def _body(x_ref, m_ref, o_ref):
    o_ref[...] = x_ref[...] * m_ref[...]


def kernel(x, mask):
    return pl.pallas_call(
        _body,
        out_shape=jax.ShapeDtypeStruct((R, C), x.dtype),
        grid=(R // BR,),
        in_specs=[
            pl.BlockSpec((BR, C), lambda j: (j, 0)),
            pl.BlockSpec((BR, 1), lambda j: (j, 0)),
        ],
        out_specs=pl.BlockSpec((BR, C), lambda j: (j, 0)),
    )(x, mask[:, None])


# speedup vs baseline: 1.1386x; 1.1386x over previous
import jax
import jax.numpy as jnp
from jax.experimental import pallas as pl

R, C = 128, 32768
BR = 64


def _body(x_ref, m_ref, o_ref):
    o_ref[...] = x_ref[...] * m_ref[...]


def kernel(x, mask):
    return pl.pallas_call(
        _body,
        out_shape=jax.ShapeDtypeStruct((R, C), x.dtype),
        grid=(R // BR,),
        in_specs=[
            pl.BlockSpec((BR, C), lambda j: (j, 0)),
            pl.BlockSpec((BR, 1), lambda j: (j, 0)),
        ],
        out_specs=pl.BlockSpec((BR, C), lambda j: (j, 0)),
    )(x, mask[:, None])
